# baseline (speedup 1.0000x reference)
import functools

import jax
import jax.numpy as jnp
from jax import lax
from jax.experimental import pallas as pl
from jax.experimental.pallas import tpu as pltpu
from jax.experimental.pallas import tpu_sc as plsc

ROWS = 128
COLS = 100000
LANES = 16
NUM_CORES = 2
NUM_SUBCORES = 16
NUM_WORKERS = NUM_CORES * NUM_SUBCORES
ROWS_PER_WORKER = ROWS // NUM_WORKERS
VECS_PER_ROW = COLS // LANES


def _sc_argmax_body(emb_hbm, out_hbm, rowbuf, outbuf):
    wid = lax.axis_index("s") * NUM_CORES + lax.axis_index("c")
    lane = lax.broadcasted_iota(jnp.int32, (LANES,), 0)
    res = jnp.zeros((LANES,), jnp.int32)

    for r in range(ROWS_PER_WORKER):
        row = wid * ROWS_PER_WORKER + r
        pltpu.sync_copy(emb_hbm.at[row], rowbuf)

        def body(i, carry):
            m, g = carry
            base = pl.multiple_of(i * LANES, LANES)
            v = rowbuf[pl.ds(base, LANES)]
            gi = lane + i * LANES
            take = v > m
            m = jnp.where(take, v, m)
            g = jnp.where(take, gi, g)
            return m, g

        m0 = jnp.full((LANES,), -jnp.inf, jnp.float32)
        m, g = lax.fori_loop(0, VECS_PER_ROW, body, (m0, lane))

        for s in (8, 4, 2, 1):
            perm = lane ^ s
            mp = m.at[perm].get(mode="promise_in_bounds", unique_indices=True)
            gp = g.at[perm].get(mode="promise_in_bounds", unique_indices=True)
            better = (mp > m) | ((mp == m) & (gp < g))
            m = jnp.where(better, mp, m)
            g = jnp.where(better, gp, g)
        res = jnp.where(lane == r, g, res)

    outbuf[...] = res
    pltpu.sync_copy(outbuf, out_hbm.at[wid])


@jax.jit
def kernel(embedding):
    call = functools.partial(
        pl.kernel,
        mesh=plsc.VectorSubcoreMesh(core_axis_name="c", subcore_axis_name="s"),
        out_type=jax.ShapeDtypeStruct((NUM_WORKERS, LANES), jnp.int32),
        scratch_types=[
            pltpu.VMEM((COLS,), jnp.float32),
            pltpu.VMEM((LANES,), jnp.int32),
        ],
    )(_sc_argmax_body)
    out = call(embedding)
    return out[:, :ROWS_PER_WORKER].reshape(ROWS)


# speedup vs baseline: 1.2108x; 1.2108x over previous
import functools

import jax
import jax.numpy as jnp
from jax import lax
from jax.experimental import pallas as pl
from jax.experimental.pallas import tpu as pltpu
from jax.experimental.pallas import tpu_sc as plsc

ROWS = 128
COLS = 100000
LANES = 16
NUM_CORES = 2
NUM_SUBCORES = 16
NUM_WORKERS = NUM_CORES * NUM_SUBCORES
ROWS_PER_WORKER = ROWS // NUM_WORKERS

CHUNK = 20000
CHUNKS_PER_ROW = COLS // CHUNK
VECS_PER_CHUNK = CHUNK // LANES
UNROLL = 10
ITERS_PER_CHUNK = VECS_PER_CHUNK // UNROLL
TOTAL_CHUNKS = ROWS_PER_WORKER * CHUNKS_PER_ROW


def _scan_chunk(buf, chunk_vec_base, lane, m, g):
    def body(i, carry):
        m, g, gvec = carry
        base = pl.multiple_of(i * (UNROLL * LANES), UNROLL * LANES)
        for u in range(UNROLL):
            v = buf[pl.ds(base + u * LANES, LANES)]
            gi = gvec + (u * LANES)
            take = v > m
            m = jnp.where(take, v, m)
            g = jnp.where(take, gi, g)
        gvec = gvec + (UNROLL * LANES)
        return m, g, gvec

    gvec0 = lane + (chunk_vec_base * LANES)
    m, g, _ = lax.fori_loop(0, ITERS_PER_CHUNK, body, (m, g, gvec0))
    return m, g


def _sc_argmax_body(emb_hbm, out_hbm, buf0, buf1, outbuf, sem0, sem1):
    wid = lax.axis_index("s") * NUM_CORES + lax.axis_index("c")
    lane = lax.broadcasted_iota(jnp.int32, (LANES,), 0)
    res = jnp.zeros((LANES,), jnp.int32)
    bufs = (buf0, buf1)
    sems = (sem0, sem1)

    chunks = [(r, c) for r in range(ROWS_PER_WORKER) for c in range(CHUNKS_PER_ROW)]

    def start(k):
        r, c = chunks[k]
        row = wid * ROWS_PER_WORKER + r
        off = pl.multiple_of(row * COLS + c * CHUNK, 8)
        return pltpu.async_copy(
            emb_hbm.at[pl.ds(off, CHUNK)], bufs[k % 2], sems[k % 2]
        )

    handles = {0: start(0)}
    m = jnp.full((LANES,), -jnp.inf, jnp.float32)
    g = lane

    for k, (r, c) in enumerate(chunks):
        if k + 1 < TOTAL_CHUNKS:
            handles[k + 1] = start(k + 1)
        handles[k].wait()
        m, g = _scan_chunk(bufs[k % 2], c * VECS_PER_CHUNK, lane, m, g)

        if c == CHUNKS_PER_ROW - 1:
            for s in (8, 4, 2, 1):
                perm = lane ^ s
                mp = m.at[perm].get(mode="promise_in_bounds", unique_indices=True)
                gp = g.at[perm].get(mode="promise_in_bounds", unique_indices=True)
                better = (mp > m) | ((mp == m) & (gp < g))
                m = jnp.where(better, mp, m)
                g = jnp.where(better, gp, g)
            res = jnp.where(lane == r, g, res)
            m = jnp.full((LANES,), -jnp.inf, jnp.float32)
            g = lane

    outbuf[...] = res
    pltpu.sync_copy(outbuf, out_hbm.at[wid])


@jax.jit
def kernel(embedding):
    call = functools.partial(
        pl.kernel,
        mesh=plsc.VectorSubcoreMesh(core_axis_name="c", subcore_axis_name="s"),
        out_type=jax.ShapeDtypeStruct((NUM_WORKERS, LANES), jnp.int32),
        scratch_types=[
            pltpu.VMEM((CHUNK,), jnp.float32),
            pltpu.VMEM((CHUNK,), jnp.float32),
            pltpu.VMEM((LANES,), jnp.int32),
            pltpu.SemaphoreType.DMA,
            pltpu.SemaphoreType.DMA,
        ],
    )(_sc_argmax_body)
    out = call(embedding.reshape(-1))
    return out[:, :ROWS_PER_WORKER].reshape(ROWS)


# speedup vs baseline: 1.2475x; 1.0303x over previous
import functools

import jax
import jax.numpy as jnp
from jax import lax
from jax.experimental import pallas as pl
from jax.experimental.pallas import tpu as pltpu
from jax.experimental.pallas import tpu_sc as plsc

ROWS = 128
COLS = 100000
LANES = 16
NUM_CORES = 2
NUM_SUBCORES = 16
NUM_WORKERS = NUM_CORES * NUM_SUBCORES
ROWS_PER_WORKER = ROWS // NUM_WORKERS

CHUNK = 20000
CHUNKS_PER_ROW = COLS // CHUNK
VECS_PER_CHUNK = CHUNK // LANES
UNROLL = 10
ITERS_PER_CHUNK = VECS_PER_CHUNK // UNROLL
TOTAL_CHUNKS = ROWS_PER_WORKER * CHUNKS_PER_ROW


def _scan_chunk(buf, chunk_vec_base, lane, ms, gs):
    def body(i, carry):
        ms, gs, gvec = carry
        base = pl.multiple_of(i * (UNROLL * LANES), UNROLL * LANES)
        ms, gs = list(ms), list(gs)
        for u in range(UNROLL):
            v = buf[pl.ds(base + u * LANES, LANES)]
            gi = gvec + (u * LANES)
            take = v > ms[u]
            ms[u] = jnp.where(take, v, ms[u])
            gs[u] = jnp.where(take, gi, gs[u])
        gvec = gvec + (UNROLL * LANES)
        return tuple(ms), tuple(gs), gvec

    gvec0 = lane + (chunk_vec_base * LANES)
    ms, gs, _ = lax.fori_loop(
        0, ITERS_PER_CHUNK, body, (tuple(ms), tuple(gs), gvec0)
    )
    return list(ms), list(gs)


def _sc_argmax_body(emb_hbm, out_hbm, buf0, buf1, outbuf, sem0, sem1):
    wid = lax.axis_index("s") * NUM_CORES + lax.axis_index("c")
    lane = lax.broadcasted_iota(jnp.int32, (LANES,), 0)
    res = jnp.zeros((LANES,), jnp.int32)
    bufs = (buf0, buf1)
    sems = (sem0, sem1)

    chunks = [(r, c) for r in range(ROWS_PER_WORKER) for c in range(CHUNKS_PER_ROW)]

    def start(k):
        r, c = chunks[k]
        row = wid * ROWS_PER_WORKER + r
        off = pl.multiple_of(row * COLS + c * CHUNK, 8)
        return pltpu.async_copy(
            emb_hbm.at[pl.ds(off, CHUNK)], bufs[k % 2], sems[k % 2]
        )

    neg_inf = jnp.full((LANES,), -jnp.inf, jnp.float32)
    handles = {0: start(0)}
    ms = [neg_inf] * UNROLL
    gs = [lane] * UNROLL

    for k, (r, c) in enumerate(chunks):
        if k + 1 < TOTAL_CHUNKS:
            handles[k + 1] = start(k + 1)
        handles[k].wait()
        ms, gs = _scan_chunk(bufs[k % 2], c * VECS_PER_CHUNK, lane, ms, gs)

        if c == CHUNKS_PER_ROW - 1:
            n = UNROLL
            while n > 1:
                h = (n + 1) // 2
                for u in range(n - h):
                    m2, g2 = ms[u + h], gs[u + h]
                    better = (m2 > ms[u]) | ((m2 == ms[u]) & (g2 < gs[u]))
                    ms[u] = jnp.where(better, m2, ms[u])
                    gs[u] = jnp.where(better, g2, gs[u])
                n = h
            m, g = ms[0], gs[0]
            for s in (8, 4, 2, 1):
                perm = lane ^ s
                mp = m.at[perm].get(mode="promise_in_bounds", unique_indices=True)
                gp = g.at[perm].get(mode="promise_in_bounds", unique_indices=True)
                better = (mp > m) | ((mp == m) & (gp < g))
                m = jnp.where(better, mp, m)
                g = jnp.where(better, gp, g)
            res = jnp.where(lane == r, g, res)
            ms = [neg_inf] * UNROLL
            gs = [lane] * UNROLL

    outbuf[...] = res
    pltpu.sync_copy(outbuf, out_hbm.at[wid])


@jax.jit
def kernel(embedding):
    call = functools.partial(
        pl.kernel,
        mesh=plsc.VectorSubcoreMesh(core_axis_name="c", subcore_axis_name="s"),
        out_type=jax.ShapeDtypeStruct((NUM_WORKERS, LANES), jnp.int32),
        scratch_types=[
            pltpu.VMEM((CHUNK,), jnp.float32),
            pltpu.VMEM((CHUNK,), jnp.float32),
            pltpu.VMEM((LANES,), jnp.int32),
            pltpu.SemaphoreType.DMA,
            pltpu.SemaphoreType.DMA,
        ],
    )(_sc_argmax_body)
    out = call(embedding.reshape(-1))
    return out[:, :ROWS_PER_WORKER].reshape(ROWS)


# speedup vs baseline: 1.2727x; 1.0202x over previous
import functools

import jax
import jax.numpy as jnp
from jax import lax
from jax.experimental import pallas as pl
from jax.experimental.pallas import tpu as pltpu
from jax.experimental.pallas import tpu_sc as plsc

ROWS = 128
COLS = 100000
LANES = 16
NUM_CORES = 2
NUM_SUBCORES = 16
NUM_WORKERS = NUM_CORES * NUM_SUBCORES
ROWS_PER_WORKER = ROWS // NUM_WORKERS

CHUNK = 20000
CHUNKS_PER_ROW = COLS // CHUNK
VECS_PER_CHUNK = CHUNK // LANES
UNROLL = 10
ITERS_PER_CHUNK = VECS_PER_CHUNK // UNROLL
TOTAL_CHUNKS = ROWS_PER_WORKER * CHUNKS_PER_ROW
NBUF = 4


def _scan_chunk(buf, chunk_vec_base, lane, ms, gs):
    def body(i, carry):
        ms, gs, gvec = carry
        base = pl.multiple_of(i * (UNROLL * LANES), UNROLL * LANES)
        ms, gs = list(ms), list(gs)
        for u in range(UNROLL):
            v = buf[pl.ds(base + u * LANES, LANES)]
            gi = gvec + (u * LANES)
            take = v > ms[u]
            ms[u] = jnp.where(take, v, ms[u])
            gs[u] = jnp.where(take, gi, gs[u])
        gvec = gvec + (UNROLL * LANES)
        return tuple(ms), tuple(gs), gvec

    gvec0 = lane + (chunk_vec_base * LANES)
    ms, gs, _ = lax.fori_loop(
        0, ITERS_PER_CHUNK, body, (tuple(ms), tuple(gs), gvec0)
    )
    return list(ms), list(gs)


def _sc_argmax_body(emb_hbm, out_hbm, *scratch):
    bufs = scratch[:NBUF]
    outbuf = scratch[NBUF]
    sems = scratch[NBUF + 1 : NBUF + 1 + NBUF]
    wid = lax.axis_index("s") * NUM_CORES + lax.axis_index("c")
    lane = lax.broadcasted_iota(jnp.int32, (LANES,), 0)
    res = jnp.zeros((LANES,), jnp.int32)

    chunks = [(r, c) for r in range(ROWS_PER_WORKER) for c in range(CHUNKS_PER_ROW)]

    def start(k):
        r, c = chunks[k]
        row = wid * ROWS_PER_WORKER + r
        off = pl.multiple_of(row * COLS + c * CHUNK, 8)
        return pltpu.async_copy(
            emb_hbm.at[pl.ds(off, CHUNK)], bufs[k % NBUF], sems[k % NBUF]
        )

    neg_inf = jnp.full((LANES,), -jnp.inf, jnp.float32)
    handles = {}
    for k in range(NBUF - 1):
        handles[k] = start(k)
    ms = [neg_inf] * UNROLL
    gs = [lane] * UNROLL

    for k, (r, c) in enumerate(chunks):
        if k + NBUF - 1 < TOTAL_CHUNKS:
            handles[k + NBUF - 1] = start(k + NBUF - 1)
        handles[k].wait()
        ms, gs = _scan_chunk(bufs[k % NBUF], c * VECS_PER_CHUNK, lane, ms, gs)

        if c == CHUNKS_PER_ROW - 1:
            n = UNROLL
            while n > 1:
                h = (n + 1) // 2
                for u in range(n - h):
                    m2, g2 = ms[u + h], gs[u + h]
                    better = (m2 > ms[u]) | ((m2 == ms[u]) & (g2 < gs[u]))
                    ms[u] = jnp.where(better, m2, ms[u])
                    gs[u] = jnp.where(better, g2, gs[u])
                n = h
            m, g = ms[0], gs[0]
            for s in (8, 4, 2, 1):
                perm = lane ^ s
                mp = m.at[perm].get(mode="promise_in_bounds", unique_indices=True)
                gp = g.at[perm].get(mode="promise_in_bounds", unique_indices=True)
                better = (mp > m) | ((mp == m) & (gp < g))
                m = jnp.where(better, mp, m)
                g = jnp.where(better, gp, g)
            res = jnp.where(lane == r, g, res)
            ms = [neg_inf] * UNROLL
            gs = [lane] * UNROLL

    outbuf[...] = res
    pltpu.sync_copy(outbuf, out_hbm.at[wid])


@jax.jit
def kernel(embedding):
    call = functools.partial(
        pl.kernel,
        mesh=plsc.VectorSubcoreMesh(core_axis_name="c", subcore_axis_name="s"),
        out_type=jax.ShapeDtypeStruct((NUM_WORKERS, LANES), jnp.int32),
        scratch_types=(
            [pltpu.VMEM((CHUNK,), jnp.float32)] * NBUF
            + [pltpu.VMEM((LANES,), jnp.int32)]
            + [pltpu.SemaphoreType.DMA] * NBUF
        ),
    )(_sc_argmax_body)
    out = call(embedding.reshape(-1))
    return out[:, :ROWS_PER_WORKER].reshape(ROWS)


# speedup vs baseline: 2.2213x; 1.7453x over previous
import functools

import jax
import jax.numpy as jnp
from jax import lax
from jax.experimental import pallas as pl
from jax.experimental.pallas import tpu as pltpu
from jax.experimental.pallas import tpu_sc as plsc

ROWS = 128
COLS = 100000
LANES = 16
NUM_CORES = 2
NUM_SUBCORES = 16

RG_ROWS = 8
NUM_RG = ROWS // RG_ROWS
HALF_TILES = 390
HALF_COLS = HALF_TILES * 128
CHUNK_TILES = 39
CHUNK_COLS = CHUNK_TILES * 128
CHUNKS_PER_HALF = HALF_TILES // CHUNK_TILES
VECS_PER_CHUNK = CHUNK_COLS // LANES
TAIL0_COL = 2 * HALF_COLS
TAIL1_COL = TAIL0_COL + 128
TAIL1_COLS = COLS - TAIL1_COL
NBUF = 2


def _scan(buf, col0, n_vecs, lane, ms, gs):
    def body(j, carry):
        ms, gs, gvec = carry
        ms, gs = list(ms), list(gs)
        base = pl.multiple_of(j * LANES, LANES)
        for u in range(RG_ROWS):
            v = buf[u, pl.ds(base, LANES)]
            take = v > ms[u]
            ms[u] = jnp.where(take, v, ms[u])
            gs[u] = jnp.where(take, gvec, gs[u])
        return tuple(ms), tuple(gs), gvec + LANES

    gvec0 = lane + col0
    ms, gs, _ = lax.fori_loop(0, n_vecs, body, (tuple(ms), tuple(gs), gvec0))
    return list(ms), list(gs)


def _sc_argmax_body(
    emb_hbm,
    out_hbm,
    buf0,
    buf1,
    tile_buf,
    tail_buf,
    stage_mg,
    partner_mg,
    out_v,
    shared_mg,
    sem0,
    sem1,
):
    cid = lax.axis_index("c")
    sid = lax.axis_index("s")
    rg = cid * (NUM_SUBCORES // 2) + sid // 2
    half = sid % 2
    row0 = pl.multiple_of(rg * RG_ROWS, RG_ROWS)
    col_base = half * HALF_COLS

    lane = lax.broadcasted_iota(jnp.int32, (LANES,), 0)
    neg_inf = jnp.full((LANES,), -jnp.inf, jnp.float32)
    bufs = (buf0, buf1)
    sems = (sem0, sem1)

    def start(k):
        coff = pl.multiple_of(col_base + k * CHUNK_COLS, 128)
        return pltpu.async_copy(
            emb_hbm.at[pl.ds(row0, RG_ROWS), pl.ds(coff, CHUNK_COLS)],
            bufs[k % NBUF],
            sems[k % NBUF],
        )

    handles = {0: start(0)}
    ms = [neg_inf] * RG_ROWS
    gs = [lane] * RG_ROWS

    for k in range(CHUNKS_PER_HALF):
        if k + 1 < CHUNKS_PER_HALF:
            handles[k + 1] = start(k + 1)
        handles[k].wait()
        ms, gs = _scan(bufs[k % NBUF], col_base + k * CHUNK_COLS, VECS_PER_CHUNK, lane, ms, gs)

    pltpu.sync_copy(
        emb_hbm.at[pl.ds(row0, RG_ROWS), pl.ds(TAIL0_COL, 128)], tile_buf
    )
    ms, gs = _scan(tile_buf, TAIL0_COL, 128 // LANES, lane, ms, gs)
    pltpu.sync_copy(
        emb_hbm.at[pl.ds(row0, RG_ROWS), pl.ds(TAIL1_COL, TAIL1_COLS)], tail_buf
    )
    ms, gs = _scan(tail_buf, TAIL1_COL, TAIL1_COLS // LANES, lane, ms, gs)

    res_m = neg_inf
    res_g = jnp.zeros((LANES,), jnp.int32)
    for u in range(RG_ROWS):
        m, g = ms[u], gs[u]
        for s in (8, 4, 2, 1):
            perm = lane ^ s
            mp = m.at[perm].get(mode="promise_in_bounds", unique_indices=True)
            gp = g.at[perm].get(mode="promise_in_bounds", unique_indices=True)
            better = (mp > m) | ((mp == m) & (gp < g))
            m = jnp.where(better, mp, m)
            g = jnp.where(better, gp, g)
        res_m = jnp.where(lane == u, m, res_m)
        res_g = jnp.where(lane == u, g, res_g)

    stage_mg[0, :] = res_m
    stage_mg[1, :] = res_g.astype(jnp.float32)
    pltpu.sync_copy(stage_mg, shared_mg.at[sid])
    plsc.subcore_barrier()
    pltpu.sync_copy(shared_mg.at[sid ^ 1], partner_mg)
    pm = partner_mg[0, :]
    pg = partner_mg[1, :].astype(jnp.int32)
    better = (pm > res_m) | ((pm == res_m) & (pg < res_g))
    merged = jnp.where(better, pg, res_g)
    out_v[...] = merged

    @pl.when(half == 0)
    def _():
        pltpu.sync_copy(out_v, out_hbm.at[rg])


@jax.jit
def kernel(embedding):
    call = functools.partial(
        pl.kernel,
        mesh=plsc.VectorSubcoreMesh(core_axis_name="c", subcore_axis_name="s"),
        out_type=jax.ShapeDtypeStruct((NUM_RG, LANES), jnp.int32),
        scratch_types=[
            pltpu.VMEM((RG_ROWS, CHUNK_COLS), jnp.float32),
            pltpu.VMEM((RG_ROWS, CHUNK_COLS), jnp.float32),
            pltpu.VMEM((RG_ROWS, 128), jnp.float32),
            pltpu.VMEM((RG_ROWS, TAIL1_COLS), jnp.float32),
            pltpu.VMEM((2, LANES), jnp.float32),
            pltpu.VMEM((2, LANES), jnp.float32),
            pltpu.VMEM((LANES,), jnp.int32),
            pltpu.VMEM_SHARED((NUM_SUBCORES, 2, LANES), jnp.float32),
            pltpu.SemaphoreType.DMA,
            pltpu.SemaphoreType.DMA,
        ],
    )(_sc_argmax_body)
    out = call(embedding)
    return out[:, :RG_ROWS].reshape(ROWS)


# speedup vs baseline: 3.9481x; 1.7774x over previous
import functools

import jax
import jax.numpy as jnp
from jax import lax
from jax.experimental import pallas as pl
from jax.experimental.pallas import tpu as pltpu
from jax.experimental.pallas import tpu_sc as plsc

ROWS = 128
SCAN = 100000
LANES = 16
NUM_CORES = 2
NUM_SUBCORES = 16
NUM_WORKERS = NUM_CORES * NUM_SUBCORES

CGROUPS = ROWS // LANES
TILE_R = 8
TR_TOTAL = SCAN // TILE_R
TR_PER_W = 391
ROWS_PER_W = TR_PER_W * TILE_R
CHUNK_TR = 23
CHUNK_ROWS = CHUNK_TR * TILE_R
NCHUNKS = TR_PER_W // CHUNK_TR
RU = 2
NBUF = 2


def _scan_chunk(buf, ms, gs, ivec):
    def body(i, carry):
        ms, gs, ivec = carry
        ms, gs = list(ms), list(gs)
        base = pl.multiple_of(i * RU, RU)
        for p in range(RU):
            for cg in range(CGROUPS):
                u = p * CGROUPS + cg
                v = buf[base + p, pl.ds(cg * LANES, LANES)]
                take = v > ms[u]
                ms[u] = jnp.where(take, v, ms[u])
                gs[u] = jnp.where(take, ivec, gs[u])
        return tuple(ms), tuple(gs), ivec + 1

    ms, gs, ivec = lax.fori_loop(
        0, CHUNK_ROWS // RU, body, (tuple(ms), tuple(gs), ivec)
    )
    return list(ms), list(gs), ivec


def _sc_argmax_body(emb_hbm, out_hbm, buf0, buf1, stage, allbuf, shared, sem0, sem1):
    cid = lax.axis_index("c")
    sid = lax.axis_index("s")
    w = cid * NUM_SUBCORES + sid
    row_base = pl.multiple_of(
        jnp.minimum(w * ROWS_PER_W, SCAN - ROWS_PER_W), TILE_R
    )

    neg_inf = jnp.full((LANES,), -jnp.inf, jnp.float32)
    zero = jnp.zeros((LANES,), jnp.int32)
    bufs = (buf0, buf1)
    sems = (sem0, sem1)

    def start(k):
        roff = pl.multiple_of(row_base + k * CHUNK_ROWS, TILE_R)
        return pltpu.async_copy(
            emb_hbm.at[pl.ds(roff, CHUNK_ROWS)], bufs[k % NBUF], sems[k % NBUF]
        )

    handles = {0: start(0)}
    nchains = CGROUPS * RU
    ms = [neg_inf] * nchains
    gs = [zero] * nchains
    ivec = zero
    for k in range(NCHUNKS):
        if k + 1 < NCHUNKS:
            handles[k + 1] = start(k + 1)
        handles[k].wait()
        ms, gs, ivec = _scan_chunk(bufs[k % NBUF], ms, gs, ivec)

    for p in range(RU):
        for cg in range(CGROUPS):
            u = p * CGROUPS + cg
            gs[u] = row_base + gs[u] * RU + p
    mm = [ms[cg] for cg in range(CGROUPS)]
    gg = [gs[cg] for cg in range(CGROUPS)]
    for p in range(1, RU):
        for cg in range(CGROUPS):
            u = p * CGROUPS + cg
            m2, g2 = ms[u], gs[u]
            better = (m2 > mm[cg]) | ((m2 == mm[cg]) & (g2 < gg[cg]))
            mm[cg] = jnp.where(better, m2, mm[cg])
            gg[cg] = jnp.where(better, g2, gg[cg])

    for cg in range(CGROUPS):
        stage[0, cg, :] = mm[cg]
        stage[1, cg, :] = gg[cg].astype(jnp.float32)
    pltpu.sync_copy(stage, shared.at[sid])
    plsc.subcore_barrier()

    @pl.when(sid == 0)
    def _():
        pltpu.sync_copy(shared, allbuf)
        fm = [allbuf[0, 0, cg, :] for cg in range(CGROUPS)]
        fg = [allbuf[0, 1, cg, :] for cg in range(CGROUPS)]
        for ww in range(1, NUM_SUBCORES):
            for cg in range(CGROUPS):
                m2 = allbuf[ww, 0, cg, :]
                g2 = allbuf[ww, 1, cg, :]
                better = (m2 > fm[cg]) | ((m2 == fm[cg]) & (g2 < fg[cg]))
                fm[cg] = jnp.where(better, m2, fm[cg])
                fg[cg] = jnp.where(better, g2, fg[cg])
        for cg in range(CGROUPS):
            stage[0, cg, :] = fm[cg]
            stage[1, cg, :] = fg[cg]
        pltpu.sync_copy(stage, out_hbm.at[cid])


@jax.jit
def kernel(embedding):
    call = functools.partial(
        pl.kernel,
        mesh=plsc.VectorSubcoreMesh(core_axis_name="c", subcore_axis_name="s"),
        out_type=jax.ShapeDtypeStruct((NUM_CORES, 2, CGROUPS, LANES), jnp.float32),
        scratch_types=[
            pltpu.VMEM((CHUNK_ROWS, ROWS), jnp.float32),
            pltpu.VMEM((CHUNK_ROWS, ROWS), jnp.float32),
            pltpu.VMEM((2, CGROUPS, LANES), jnp.float32),
            pltpu.VMEM((NUM_SUBCORES, 2, CGROUPS, LANES), jnp.float32),
            pltpu.VMEM_SHARED((NUM_SUBCORES, 2, CGROUPS, LANES), jnp.float32),
            pltpu.SemaphoreType.DMA,
            pltpu.SemaphoreType.DMA,
        ],
    )(_sc_argmax_body)
    out = call(embedding.T)
    m = out[:, 0].reshape(NUM_CORES, ROWS)
    g = out[:, 1].reshape(NUM_CORES, ROWS)
    take1 = (m[1] > m[0]) | ((m[1] == m[0]) & (g[1] < g[0]))
    return jnp.where(take1, g[1], g[0]).astype(jnp.int32)


# speedup vs baseline: 4.4953x; 1.1386x over previous
import functools

import jax
import jax.numpy as jnp
from jax import lax
from jax.experimental import pallas as pl
from jax.experimental.pallas import tpu as pltpu
from jax.experimental.pallas import tpu_sc as plsc

ROWS = 128
SCAN = 100000
LANES = 16
NUM_CORES = 2
NUM_SUBCORES = 16
NUM_WORKERS = NUM_CORES * NUM_SUBCORES

CGROUPS = ROWS // LANES
TILE_R = 8
CHUNK_TR = 23
CHUNK_ROWS = CHUNK_TR * TILE_R
NCHUNKS = 6
TR_PER_W = CHUNK_TR * NCHUNKS
ROWS_PER_W = TR_PER_W * TILE_R
SC_SPAN = NUM_WORKERS * ROWS_PER_W
RU = 2
NBUF = 2

TC_BLOCK = 4000
TC_START_BLK = 8
TC_NBLKS = SCAN // TC_BLOCK - TC_START_BLK


def _scan_chunk(buf, ms, gs, ivec):
    def body(i, carry):
        ms, gs, ivec = carry
        ms, gs = list(ms), list(gs)
        base = pl.multiple_of(i * RU, RU)
        for p in range(RU):
            for cg in range(CGROUPS):
                u = p * CGROUPS + cg
                v = buf[base + p, pl.ds(cg * LANES, LANES)]
                take = v > ms[u]
                ms[u] = jnp.where(take, v, ms[u])
                gs[u] = jnp.where(take, ivec, gs[u])
        return tuple(ms), tuple(gs), ivec + 1

    ms, gs, ivec = lax.fori_loop(
        0, CHUNK_ROWS // RU, body, (tuple(ms), tuple(gs), ivec)
    )
    return list(ms), list(gs), ivec


def _sc_argmax_body(emb_hbm, out_hbm, buf0, buf1, stage, allbuf, shared, sem0, sem1):
    cid = lax.axis_index("c")
    sid = lax.axis_index("s")
    w = cid * NUM_SUBCORES + sid
    row_base = pl.multiple_of(w * ROWS_PER_W, TILE_R)

    neg_inf = jnp.full((LANES,), -jnp.inf, jnp.float32)
    zero = jnp.zeros((LANES,), jnp.int32)
    bufs = (buf0, buf1)
    sems = (sem0, sem1)

    def start(k):
        roff = pl.multiple_of(row_base + k * CHUNK_ROWS, TILE_R)
        return pltpu.async_copy(
            emb_hbm.at[pl.ds(roff, CHUNK_ROWS)], bufs[k % NBUF], sems[k % NBUF]
        )

    handles = {0: start(0)}
    nchains = CGROUPS * RU
    ms = [neg_inf] * nchains
    gs = [zero] * nchains
    ivec = zero
    for k in range(NCHUNKS):
        if k + 1 < NCHUNKS:
            handles[k + 1] = start(k + 1)
        handles[k].wait()
        ms, gs, ivec = _scan_chunk(bufs[k % NBUF], ms, gs, ivec)

    for p in range(RU):
        for cg in range(CGROUPS):
            u = p * CGROUPS + cg
            gs[u] = row_base + gs[u] * RU + p
    mm = [ms[cg] for cg in range(CGROUPS)]
    gg = [gs[cg] for cg in range(CGROUPS)]
    for p in range(1, RU):
        for cg in range(CGROUPS):
            u = p * CGROUPS + cg
            m2, g2 = ms[u], gs[u]
            better = (m2 > mm[cg]) | ((m2 == mm[cg]) & (g2 < gg[cg]))
            mm[cg] = jnp.where(better, m2, mm[cg])
            gg[cg] = jnp.where(better, g2, gg[cg])

    for cg in range(CGROUPS):
        stage[0, cg, :] = mm[cg]
        stage[1, cg, :] = gg[cg].astype(jnp.float32)
    pltpu.sync_copy(stage, shared.at[sid])
    plsc.subcore_barrier()

    @pl.when(sid == 0)
    def _():
        pltpu.sync_copy(shared, allbuf)
        fm = [allbuf[0, 0, cg, :] for cg in range(CGROUPS)]
        fg = [allbuf[0, 1, cg, :] for cg in range(CGROUPS)]
        for ww in range(1, NUM_SUBCORES):
            for cg in range(CGROUPS):
                m2 = allbuf[ww, 0, cg, :]
                g2 = allbuf[ww, 1, cg, :]
                better = (m2 > fm[cg]) | ((m2 == fm[cg]) & (g2 < fg[cg]))
                fm[cg] = jnp.where(better, m2, fm[cg])
                fg[cg] = jnp.where(better, g2, fg[cg])
        for cg in range(CGROUPS):
            stage[0, cg, :] = fm[cg]
            stage[1, cg, :] = fg[cg]
        pltpu.sync_copy(stage, out_hbm.at[cid])


def _tc_argmax_body(x_ref, out_ref):
    i = pl.program_id(0)
    x = x_ref[...]
    m = jnp.max(x, axis=0)
    ridx = lax.broadcasted_iota(jnp.int32, (TC_BLOCK, ROWS), 0)
    g = jnp.min(jnp.where(x == m[None, :], ridx, jnp.int32(0x7FFFFFFF)), axis=0)
    row0 = (TC_START_BLK + i) * TC_BLOCK
    gf = (g + row0).astype(jnp.float32)

    @pl.when(i == 0)
    def _():
        out_ref[0, :] = m
        out_ref[1, :] = gf

    @pl.when(i > 0)
    def _():
        pm = out_ref[0, :]
        better = m > pm
        out_ref[0, :] = jnp.where(better, m, pm)
        out_ref[1, :] = jnp.where(better, gf, out_ref[1, :])


@jax.jit
def kernel(embedding):
    emb_t = embedding.T

    sc_call = functools.partial(
        pl.kernel,
        mesh=plsc.VectorSubcoreMesh(core_axis_name="c", subcore_axis_name="s"),
        out_type=jax.ShapeDtypeStruct((NUM_CORES, 2, CGROUPS, LANES), jnp.float32),
        scratch_types=[
            pltpu.VMEM((CHUNK_ROWS, ROWS), jnp.float32),
            pltpu.VMEM((CHUNK_ROWS, ROWS), jnp.float32),
            pltpu.VMEM((2, CGROUPS, LANES), jnp.float32),
            pltpu.VMEM((NUM_SUBCORES, 2, CGROUPS, LANES), jnp.float32),
            pltpu.VMEM_SHARED((NUM_SUBCORES, 2, CGROUPS, LANES), jnp.float32),
            pltpu.SemaphoreType.DMA,
            pltpu.SemaphoreType.DMA,
        ],
    )(_sc_argmax_body)
    sc_out = sc_call(emb_t)

    tc_out = pl.pallas_call(
        _tc_argmax_body,
        grid=(TC_NBLKS,),
        in_specs=[
            pl.BlockSpec((TC_BLOCK, ROWS), lambda i: (TC_START_BLK + i, 0))
        ],
        out_specs=pl.BlockSpec((2, ROWS), lambda i: (0, 0)),
        out_shape=jax.ShapeDtypeStruct((2, ROWS), jnp.float32),
        compiler_params=pltpu.CompilerParams(
            dimension_semantics=("arbitrary",)
        ),
    )(emb_t)

    m0, g0 = sc_out[0, 0].reshape(ROWS), sc_out[0, 1].reshape(ROWS)
    m1, g1 = sc_out[1, 0].reshape(ROWS), sc_out[1, 1].reshape(ROWS)
    take1 = (m1 > m0) | ((m1 == m0) & (g1 < g0))
    m = jnp.where(take1, m1, m0)
    g = jnp.where(take1, g1, g0)
    mt, gt = tc_out[0], tc_out[1]
    taket = (mt > m) | ((mt == m) & (gt < g))
    return jnp.where(taket, gt, g).astype(jnp.int32)


# speedup vs baseline: 4.7911x; 1.0658x over previous
import functools

import jax
import jax.numpy as jnp
from jax import lax
from jax.experimental import pallas as pl
from jax.experimental.pallas import tpu as pltpu
from jax.experimental.pallas import tpu_sc as plsc

ROWS = 128
SCAN = 100000
LANES = 16
NUM_CORES = 2
NUM_SUBCORES = 16
NUM_WORKERS = NUM_CORES * NUM_SUBCORES

CGROUPS = ROWS // LANES
TILE_R = 8
CHUNK_TR = 46
CHUNK_ROWS = CHUNK_TR * TILE_R
NCHUNKS = 4
TR_PER_W = CHUNK_TR * NCHUNKS
ROWS_PER_W = TR_PER_W * TILE_R
SC_SPAN = NUM_WORKERS * ROWS_PER_W
RU = 2
NBUF = 2

TC_BLOCK = 4000
TC_START_BLK = 11
TC_NBLKS = SCAN // TC_BLOCK - TC_START_BLK


def _scan_chunk(buf, ms, gs, ivec):
    def body(i, carry):
        ms, gs, ivec = carry
        ms, gs = list(ms), list(gs)
        base = pl.multiple_of(i * RU, RU)
        for p in range(RU):
            for cg in range(CGROUPS):
                u = p * CGROUPS + cg
                v = buf[base + p, pl.ds(cg * LANES, LANES)]
                take = v > ms[u]
                ms[u] = jnp.where(take, v, ms[u])
                gs[u] = jnp.where(take, ivec, gs[u])
        return tuple(ms), tuple(gs), ivec + 1

    ms, gs, ivec = lax.fori_loop(
        0, CHUNK_ROWS // RU, body, (tuple(ms), tuple(gs), ivec)
    )
    return list(ms), list(gs), ivec


def _sc_argmax_body(emb_hbm, out_hbm, buf0, buf1, stage, allbuf, shared, sem0, sem1):
    cid = lax.axis_index("c")
    sid = lax.axis_index("s")
    w = cid * NUM_SUBCORES + sid
    row_base = pl.multiple_of(w * ROWS_PER_W, TILE_R)

    neg_inf = jnp.full((LANES,), -jnp.inf, jnp.float32)
    zero = jnp.zeros((LANES,), jnp.int32)
    bufs = (buf0, buf1)
    sems = (sem0, sem1)

    def start(k):
        roff = pl.multiple_of(row_base + k * CHUNK_ROWS, TILE_R)
        return pltpu.async_copy(
            emb_hbm.at[pl.ds(roff, CHUNK_ROWS)], bufs[k % NBUF], sems[k % NBUF]
        )

    handles = {0: start(0)}
    nchains = CGROUPS * RU
    ms = [neg_inf] * nchains
    gs = [zero] * nchains
    ivec = zero
    for k in range(NCHUNKS):
        if k + 1 < NCHUNKS:
            handles[k + 1] = start(k + 1)
        handles[k].wait()
        ms, gs, ivec = _scan_chunk(bufs[k % NBUF], ms, gs, ivec)

    for p in range(RU):
        for cg in range(CGROUPS):
            u = p * CGROUPS + cg
            gs[u] = row_base + gs[u] * RU + p
    mm = [ms[cg] for cg in range(CGROUPS)]
    gg = [gs[cg] for cg in range(CGROUPS)]
    for p in range(1, RU):
        for cg in range(CGROUPS):
            u = p * CGROUPS + cg
            m2, g2 = ms[u], gs[u]
            better = (m2 > mm[cg]) | ((m2 == mm[cg]) & (g2 < gg[cg]))
            mm[cg] = jnp.where(better, m2, mm[cg])
            gg[cg] = jnp.where(better, g2, gg[cg])

    for cg in range(CGROUPS):
        stage[0, cg, :] = mm[cg]
        stage[1, cg, :] = gg[cg].astype(jnp.float32)
    pltpu.sync_copy(stage, shared.at[sid])
    plsc.subcore_barrier()

    @pl.when(sid == 0)
    def _():
        pltpu.sync_copy(shared, allbuf)
        fm = [allbuf[0, 0, cg, :] for cg in range(CGROUPS)]
        fg = [allbuf[0, 1, cg, :] for cg in range(CGROUPS)]
        for ww in range(1, NUM_SUBCORES):
            for cg in range(CGROUPS):
                m2 = allbuf[ww, 0, cg, :]
                g2 = allbuf[ww, 1, cg, :]
                better = (m2 > fm[cg]) | ((m2 == fm[cg]) & (g2 < fg[cg]))
                fm[cg] = jnp.where(better, m2, fm[cg])
                fg[cg] = jnp.where(better, g2, fg[cg])
        for cg in range(CGROUPS):
            stage[0, cg, :] = fm[cg]
            stage[1, cg, :] = fg[cg]
        pltpu.sync_copy(stage, out_hbm.at[cid])


def _tc_argmax_body(x_ref, out_ref):
    i = pl.program_id(0)
    x = x_ref[...]
    m = jnp.max(x, axis=0)
    ridx = lax.broadcasted_iota(jnp.int32, (TC_BLOCK, ROWS), 0)
    g = jnp.min(jnp.where(x == m[None, :], ridx, jnp.int32(0x7FFFFFFF)), axis=0)
    row0 = (TC_START_BLK + i) * TC_BLOCK
    gf = (g + row0).astype(jnp.float32)

    @pl.when(i == 0)
    def _():
        out_ref[0, :] = m
        out_ref[1, :] = gf

    @pl.when(i > 0)
    def _():
        pm = out_ref[0, :]
        better = m > pm
        out_ref[0, :] = jnp.where(better, m, pm)
        out_ref[1, :] = jnp.where(better, gf, out_ref[1, :])


@jax.jit
def kernel(embedding):
    emb_t = embedding.T

    sc_call = functools.partial(
        pl.kernel,
        mesh=plsc.VectorSubcoreMesh(core_axis_name="c", subcore_axis_name="s"),
        out_type=jax.ShapeDtypeStruct((NUM_CORES, 2, CGROUPS, LANES), jnp.float32),
        scratch_types=[
            pltpu.VMEM((CHUNK_ROWS, ROWS), jnp.float32),
            pltpu.VMEM((CHUNK_ROWS, ROWS), jnp.float32),
            pltpu.VMEM((2, CGROUPS, LANES), jnp.float32),
            pltpu.VMEM((NUM_SUBCORES, 2, CGROUPS, LANES), jnp.float32),
            pltpu.VMEM_SHARED((NUM_SUBCORES, 2, CGROUPS, LANES), jnp.float32),
            pltpu.SemaphoreType.DMA,
            pltpu.SemaphoreType.DMA,
        ],
    )(_sc_argmax_body)
    sc_out = sc_call(emb_t)

    tc_out = pl.pallas_call(
        _tc_argmax_body,
        grid=(TC_NBLKS,),
        in_specs=[
            pl.BlockSpec((TC_BLOCK, ROWS), lambda i: (TC_START_BLK + i, 0))
        ],
        out_specs=pl.BlockSpec((2, ROWS), lambda i: (0, 0)),
        out_shape=jax.ShapeDtypeStruct((2, ROWS), jnp.float32),
        compiler_params=pltpu.CompilerParams(
            dimension_semantics=("arbitrary",)
        ),
    )(emb_t)

    m0, g0 = sc_out[0, 0].reshape(ROWS), sc_out[0, 1].reshape(ROWS)
    m1, g1 = sc_out[1, 0].reshape(ROWS), sc_out[1, 1].reshape(ROWS)
    take1 = (m1 > m0) | ((m1 == m0) & (g1 < g0))
    m = jnp.where(take1, m1, m0)
    g = jnp.where(take1, g1, g0)
    mt, gt = tc_out[0], tc_out[1]
    taket = (mt > m) | ((mt == m) & (gt < g))
    return jnp.where(taket, gt, g).astype(jnp.int32)
